# baseline (speedup 1.0000x reference)
import functools

import jax
import jax.numpy as jnp
from jax import lax
from jax.experimental import pallas as pl
from jax.experimental.pallas import tpu as pltpu
from jax.experimental.pallas import tpu_sc as plsc

_ALPHA_C = 1.0
_QPW = 512
_NQV = _QPW // 16


def _sc_chamfer(B, V2, V1):
    VT = V2 + V1
    mesh = plsc.VectorSubcoreMesh(core_axis_name="c", subcore_axis_name="s")

    @functools.partial(
        pl.kernel, mesh=mesh,
        out_type=jax.ShapeDtypeStruct((B, VT), jnp.float32),
        scratch_types=[
            pltpu.VMEM((3, 2048), jnp.float32),
            pltpu.VMEM((2048,), jnp.float32),
            pltpu.VMEM((3, _QPW), jnp.float32),
            pltpu.VMEM((_QPW,), jnp.float32),
            pltpu.VMEM((_QPW,), jnp.float32),
        ],
    )
    def body(ptsf_hbm, n2_hbm, out_hbm, keys_v, k2_v, q_v, q2_v, res_v):
        w = lax.axis_index("c") * 16 + lax.axis_index("s")
        task = w // 4
        quarter = w % 4
        b = task % B
        dirn = task // B
        qside = dirn * V2
        kside = (1 - dirn) * V2
        qoff = qside + quarter * _QPW

        pltpu.sync_copy(ptsf_hbm.at[b, :, pl.ds(kside, 2048)], keys_v)
        pltpu.sync_copy(n2_hbm.at[b, pl.ds(kside, 2048)], k2_v)
        pltpu.sync_copy(ptsf_hbm.at[b, :, pl.ds(qoff, _QPW)], q_v)
        pltpu.sync_copy(n2_hbm.at[b, pl.ds(qoff, _QPW)], q2_v)

        for g0 in range(0, _NQV, 4):
            qs = []
            for g in range(g0, g0 + 4):
                qs.append((q_v[0, pl.ds(g * 16, 16)],
                           q_v[1, pl.ds(g * 16, 16)],
                           q_v[2, pl.ds(g * 16, 16)]))
            inf16 = jnp.full((16,), jnp.inf, jnp.float32)

            def kstep(jc, accs):
                base = jc * 16
                kxv = keys_v[0, pl.ds(base, 16)]
                kyv = keys_v[1, pl.ds(base, 16)]
                kzv = keys_v[2, pl.ds(base, 16)]
                k2v = k2_v[pl.ds(base, 16)]
                for i in range(16):
                    a = kxv[i] * -2.0
                    bb = kyv[i] * -2.0
                    c = kzv[i] * -2.0
                    e = k2v[i]
                    new = []
                    for (qx, qy, qz), acc in zip(qs, accs):
                        d = qx * a + (qy * bb + (qz * c + e))
                        new.append(jnp.minimum(acc, d))
                    accs = tuple(new)
                return accs

            accs = lax.fori_loop(0, 128, kstep,
                                 (inf16, inf16, inf16, inf16))
            for g, acc in zip(range(g0, g0 + 4), accs):
                q2g = q2_v[pl.ds(g * 16, 16)]
                res_v[pl.ds(g * 16, 16)] = jnp.maximum(acc + q2g, 0.0)

        pltpu.sync_copy(res_v, out_hbm.at[b, pl.ds(qoff, _QPW)])

    return body


def _epilogue_body(cham_ref, mask_ref, conf_ref, conf_out, pred_out, gt_out):
    V2 = mask_ref.shape[2]
    lp = jnp.sqrt(cham_ref[:, :, :V2]) * 100.0
    lg = jnp.sqrt(cham_ref[:, :, V2:]) * 100.0
    m = mask_ref[...]
    c = conf_ref[...]
    pred_out[...] = lp * m
    conf_out[...] = (lp * c - _ALPHA_C * jnp.log(c)) * m
    gt_out[...] = lg


def kernel(x_gt, x_pred, mask, confidence):
    B, V1, _ = x_gt.shape
    V2 = x_pred.shape[1]
    VT = V2 + V1
    xp = x_pred * mask[..., None]
    pts = jnp.concatenate([xp, x_gt], axis=1)
    n2 = jnp.sum(pts * pts, axis=2)
    u = lax.bitcast_convert_type(pts, jnp.uint32)
    u = (u + jnp.uint32(0x7FFF) + ((u >> 16) & jnp.uint32(1))) \
        & jnp.uint32(0xFFFF0000)
    ptsf = jnp.transpose(lax.bitcast_convert_type(u, jnp.float32),
                         (0, 2, 1))

    cham = _sc_chamfer(B, V2, V1)(ptsf, n2)

    mask3 = mask.reshape(B, 1, V2)
    conf3 = confidence.reshape(B, 1, V2)
    full = lambda i: (0, 0, 0)
    conf_o, pred_o, gt_o = pl.pallas_call(
        _epilogue_body,
        grid=(1,),
        in_specs=[
            pl.BlockSpec((B, 1, VT), full),
            pl.BlockSpec((B, 1, V2), full),
            pl.BlockSpec((B, 1, V2), full),
        ],
        out_specs=[
            pl.BlockSpec((B, 1, V2), full),
            pl.BlockSpec((B, 1, V2), full),
            pl.BlockSpec((B, 1, V1), full),
        ],
        out_shape=[
            jax.ShapeDtypeStruct((B, 1, V2), jnp.float32),
            jax.ShapeDtypeStruct((B, 1, V2), jnp.float32),
            jax.ShapeDtypeStruct((B, 1, V1), jnp.float32),
        ],
    )(cham.reshape(B, 1, VT), mask3, conf3)

    return (conf_o.reshape(B, V2), pred_o.reshape(B, V2), gt_o.reshape(B, V1))


# speedup vs baseline: 26.7784x; 26.7784x over previous
import jax
import jax.numpy as jnp
from jax import lax
from jax.experimental import pallas as pl
from jax.experimental.pallas import tpu as pltpu

_MC = 512
_ALPHA_C = 1.0


def _split3(v):
    h = v.astype(jnp.bfloat16)
    r = v - h.astype(jnp.float32)
    l = r.astype(jnp.bfloat16)
    l2 = (r - l.astype(jnp.float32)).astype(jnp.bfloat16)
    return h, l, l2


def _chamfer_body(pts_ref, n2_ref, mask_ref, conf_ref,
                  conf_out, pred_out, gt_out, minacc):
    B = pts_ref.shape[0]
    VT = pts_ref.shape[2]
    V2 = mask_ref.shape[2]
    V1 = VT - V2
    ones3x = jnp.ones((3, V2), jnp.bfloat16)
    ones3y = jnp.ones((3, V1), jnp.bfloat16)
    zeros7x = jnp.zeros((7, V2), jnp.bfloat16)
    zeros7y = jnp.zeros((7, V1), jnp.bfloat16)
    dn = (((0,), (0,)), ((), ()))

    for b in range(B):
        m = mask_ref[b]

        xh, xl, xl2 = _split3(n2_ref[b, :, :V2])
        x_rhs = jnp.concatenate(
            [-2.0 * pts_ref[b, :, :V2], ones3x, xh, xl, xl2, zeros7x], 0)

        yh, yl, yl2 = _split3(n2_ref[b, :, V2:])
        y_lhs = jnp.concatenate(
            [pts_ref[b, :, V2:], yh, yl, yl2, ones3y, zeros7y], 0)

        for j in range(V2 // _MC):
            sl = slice(j * _MC, (j + 1) * _MC)
            dj = lax.dot_general(y_lhs, x_rhs[:, sl], dn,
                                 preferred_element_type=jnp.float32)
            cmin = jnp.maximum(jnp.min(dj, axis=0, keepdims=True), 0.0)
            lp = jnp.sqrt(cmin) * 100.0
            mj = m[:, sl]
            cj = conf_ref[b, :, sl]
            pred_out[b, :, sl] = lp * mj
            conf_out[b, :, sl] = (lp * cj - _ALPHA_C * jnp.log(cj)) * mj
            if j == 0:
                minacc[...] = dj
            else:
                minacc[...] = jnp.minimum(minacc[...], dj)

        rmin = jnp.maximum(
            jnp.min(minacc[...], axis=1, keepdims=True), 0.0)
        gt_out[b] = jnp.transpose(jnp.sqrt(rmin) * 100.0, (1, 0))


def kernel(x_gt, x_pred, mask, confidence):
    B, V1, _ = x_gt.shape
    V2 = x_pred.shape[1]
    xp = x_pred * mask[..., None]
    pts = jnp.concatenate([xp, x_gt], axis=1)
    n2 = jnp.sum(pts * pts, axis=2).reshape(B, 1, V2 + V1)
    pts_t = jnp.transpose(pts.astype(jnp.bfloat16), (0, 2, 1))
    mask3 = mask.reshape(B, 1, V2)
    conf3 = confidence.reshape(B, 1, V2)

    full = lambda i: (0, 0, 0)
    conf_o, pred_o, gt_o = pl.pallas_call(
        _chamfer_body,
        grid=(1,),
        in_specs=[
            pl.BlockSpec((B, 3, V2 + V1), full),
            pl.BlockSpec((B, 1, V2 + V1), full),
            pl.BlockSpec((B, 1, V2), full),
            pl.BlockSpec((B, 1, V2), full),
        ],
        out_specs=[
            pl.BlockSpec((B, 1, V2), full),
            pl.BlockSpec((B, 1, V2), full),
            pl.BlockSpec((B, 1, V1), full),
        ],
        out_shape=[
            jax.ShapeDtypeStruct((B, 1, V2), jnp.float32),
            jax.ShapeDtypeStruct((B, 1, V2), jnp.float32),
            jax.ShapeDtypeStruct((B, 1, V1), jnp.float32),
        ],
        scratch_shapes=[pltpu.VMEM((V1, _MC), jnp.float32)],
    )(pts_t, n2, mask3, conf3)

    return (conf_o.reshape(B, V2), pred_o.reshape(B, V2), gt_o.reshape(B, V1))
